# baseline (speedup 1.0000x reference)
import functools

import jax
import jax.numpy as jnp
from jax import lax
from jax.experimental import pallas as pl
from jax.experimental.pallas import tpu as pltpu
from jax.experimental.pallas import tpu_sc as plsc

EMB_DIM = 64
_NC = 2
_NS = 16
_NW = _NC * _NS


@functools.lru_cache(maxsize=None)
def _make_gather(n_total, chunk):
    b_per_w = n_total // _NW
    t = b_per_w // chunk
    mesh = plsc.VectorSubcoreMesh(core_axis_name="c", subcore_axis_name="s")

    @functools.partial(
        pl.kernel,
        mesh=mesh,
        out_type=jax.ShapeDtypeStruct((n_total, EMB_DIM), jnp.float32),
        compiler_params=pltpu.CompilerParams(use_tc_tiling_on_sc=False),
        scratch_types=[
            pltpu.VMEM((chunk,), jnp.int32),
            pltpu.VMEM((chunk,), jnp.int32),
            pltpu.VMEM((2, chunk, EMB_DIM), jnp.float32),
            pltpu.SemaphoreType.DMA,
            pltpu.SemaphoreType.DMA,
            pltpu.SemaphoreType.DMA,
        ],
    )
    def gather_kernel(idx_hbm, table_hbm, out_hbm,
                      idx_v0, idx_v1, rows_v, isem, gsem, ssem):
        wid = lax.axis_index("s") * _NC + lax.axis_index("c")
        idx_bufs = [idx_v0, idx_v1]
        ic = [None] * t
        gc = [None] * t
        sc = [None] * t
        ic[0] = pltpu.async_copy(idx_hbm.at[wid * t + 0], idx_bufs[0], isem)
        if t > 1:
            ic[1] = pltpu.async_copy(idx_hbm.at[wid * t + 1], idx_bufs[1], isem)
        ic[0].wait()
        gc[0] = pltpu.async_copy(table_hbm.at[idx_bufs[0]], rows_v.at[0], gsem)
        for c in range(t):
            p = c % 2
            if c + 1 < t:
                ic[c + 1].wait()
                if c >= 1:
                    sc[c - 1].wait()
                gc[c + 1] = pltpu.async_copy(
                    table_hbm.at[idx_bufs[(c + 1) % 2]],
                    rows_v.at[(c + 1) % 2], gsem)
            gc[c].wait()
            if c + 2 < t:
                ic[c + 2] = pltpu.async_copy(
                    idx_hbm.at[wid * t + c + 2], idx_bufs[p], isem)
            sc[c] = pltpu.async_copy(
                rows_v.at[p],
                out_hbm.at[pl.ds((wid * t + c) * chunk, chunk)], ssem)
        if t > 1:
            sc[t - 2].wait()
        sc[t - 1].wait()

    return gather_kernel


def kernel(inp, emb_weight):
    b, s = inp.shape
    n_total = b * s
    chunk = 640
    idx = inp.reshape(n_total // chunk, chunk)
    out = _make_gather(n_total, chunk)(idx, emb_weight)
    return out.reshape(b, s, EMB_DIM)
